# baseline (speedup 1.0000x reference)
import functools

import jax
import jax.numpy as jnp
import numpy as np
from jax import lax
from jax.experimental import pallas as pl
from jax.experimental.pallas import tpu as pltpu
from jax.experimental.pallas import tpu_sc as plsc

_NF = 9
_EMB = 128
_N = 100000
_CHUNK = 160
_NCHUNKS = _N // _CHUNK
_NW = 32
_L = 16


def _tec_body(xt_hbm, tbl_hbm, out_hbm, xi_v, code_v, rows_v, sem):
    wid = lax.axis_index("s") * 2 + lax.axis_index("c")
    nchunks = (_NCHUNKS - wid + _NW - 1) // _NW

    def chunk_body(k, _):
        cid = wid + k * _NW
        base = cid * _CHUNK
        for i in range(_NF):
            pltpu.sync_copy(
                xt_hbm.at[pl.ds(i * _N + base, _CHUNK)], xi_v.at[pl.ds(i * _CHUNK, _CHUNK)]
            )
        for j in range(_CHUNK // _L):
            code = jnp.zeros((_L,), jnp.int32)
            for i in range(_NF):
                code = code + (xi_v[pl.ds(i * _CHUNK + j * _L, _L)] << i)
            code_v[pl.ds(j * _L, _L)] = code
        pltpu.async_copy(tbl_hbm.at[code_v], rows_v, sem).wait()
        pltpu.sync_copy(rows_v, out_hbm.at[pl.ds(base, _CHUNK)])
        return ()

    lax.fori_loop(0, nchunks, chunk_body, (), unroll=False)


@jax.jit
def _run_sc(xt, tbl):
    mesh = plsc.VectorSubcoreMesh(core_axis_name="c", subcore_axis_name="s")
    kfn = functools.partial(
        pl.kernel,
        mesh=mesh,
        out_type=jax.ShapeDtypeStruct((_N, _EMB), jnp.float32),
        scratch_types=[
            pltpu.VMEM((_NF * _CHUNK,), jnp.int32),
            pltpu.VMEM((_CHUNK,), jnp.int32),
            pltpu.VMEM((_CHUNK, _EMB), jnp.float32),
            pltpu.SemaphoreType.DMA,
        ],
    )(_tec_body)
    return kfn(xt, tbl)


def kernel(x, W0, W1, W2, W3, W4, W5, W6, W7, W8):
    xt = x.astype(jnp.int32).T.reshape(-1)
    tables = [W0, W1, W2, W3, W4, W5, W6, W7, W8]
    codes = np.arange(512)
    tbl = jnp.zeros((512, _EMB), jnp.float32)
    for i, w in enumerate(tables):
        bits = jnp.asarray((codes >> i) & 1, dtype=jnp.int32)
        tbl = tbl + w[bits, :]
    return _run_sc(xt, tbl)


# speedup vs baseline: 1.3732x; 1.3732x over previous
import functools

import jax
import jax.numpy as jnp
import numpy as np
from jax import lax
from jax.experimental import pallas as pl
from jax.experimental.pallas import tpu as pltpu
from jax.experimental.pallas import tpu_sc as plsc

_NF = 9
_EMB = 128
_N = 100000
_CHUNK = 400
_NCHUNKS = _N // _CHUNK
_NW = 32
_L = 16


def _tec_body(xt_hbm, tbl_hbm, out_hbm, xi_v, code_v, rows_v, sem):
    wid = lax.axis_index("s") * 2 + lax.axis_index("c")
    nchunks = (_NCHUNKS - wid + _NW - 1) // _NW

    def chunk_body(k, _):
        cid = wid + k * _NW
        base = cid * _CHUNK
        for i in range(_NF):
            pltpu.sync_copy(
                xt_hbm.at[pl.ds(i * _N + base, _CHUNK)], xi_v.at[pl.ds(i * _CHUNK, _CHUNK)]
            )
        for j in range(_CHUNK // _L):
            code = jnp.zeros((_L,), jnp.int32)
            for i in range(_NF):
                code = code + (xi_v[pl.ds(i * _CHUNK + j * _L, _L)] << i)
            code_v[pl.ds(j * _L, _L)] = code
        pltpu.async_copy(tbl_hbm.at[code_v], rows_v, sem).wait()
        pltpu.sync_copy(rows_v, out_hbm.at[pl.ds(base, _CHUNK)])
        return ()

    lax.fori_loop(0, nchunks, chunk_body, (), unroll=False)


@jax.jit
def _run_sc(xt, tbl):
    mesh = plsc.VectorSubcoreMesh(core_axis_name="c", subcore_axis_name="s")
    kfn = functools.partial(
        pl.kernel,
        mesh=mesh,
        out_type=jax.ShapeDtypeStruct((_N, _EMB), jnp.float32),
        scratch_types=[
            pltpu.VMEM((_NF * _CHUNK,), jnp.int32),
            pltpu.VMEM((_CHUNK,), jnp.int32),
            pltpu.VMEM((_CHUNK, _EMB), jnp.float32),
            pltpu.SemaphoreType.DMA,
        ],
    )(_tec_body)
    return kfn(xt, tbl)


def kernel(x, W0, W1, W2, W3, W4, W5, W6, W7, W8):
    xt = x.astype(jnp.int32).T.reshape(-1)
    tables = [W0, W1, W2, W3, W4, W5, W6, W7, W8]
    codes = np.arange(512)
    tbl = jnp.zeros((512, _EMB), jnp.float32)
    for i, w in enumerate(tables):
        bits = jnp.asarray((codes >> i) & 1, dtype=jnp.int32)
        tbl = tbl + w[bits, :]
    return _run_sc(xt, tbl)
